# baseline (speedup 1.0000x reference)
import functools

import jax
import jax.numpy as jnp
from jax.experimental import pallas as pl

N = 10000
D_IN = 128
D_OUT = 128
BM = 200


def _linear_kernel(x_ref, w_ref, b_ref, h_ref):
    h_ref[...] = jax.lax.dot_general(
        x_ref[...], w_ref[...],
        (((1,), (1,)), ((), ())),
        preferred_element_type=jnp.float32,
    ) + b_ref[...]


def _agg_kernel(adj_ref, h_ref, out_ref):
    out_ref[...] = jnp.dot(
        adj_ref[...], h_ref[...], preferred_element_type=jnp.float32
    )


@jax.jit
def kernel(x, adj, W, b):
    b2 = b.reshape(1, D_OUT)
    h = pl.pallas_call(
        _linear_kernel,
        out_shape=jax.ShapeDtypeStruct((N, D_OUT), jnp.float32),
    )(x, W, b2)

    grid = (N // BM,)
    out = pl.pallas_call(
        _agg_kernel,
        grid=grid,
        in_specs=[
            pl.BlockSpec((BM, N), lambda i: (i, 0)),
            pl.BlockSpec((N, D_OUT), lambda i: (0, 0)),
        ],
        out_specs=pl.BlockSpec((BM, D_OUT), lambda i: (i, 0)),
        out_shape=jax.ShapeDtypeStruct((N, D_OUT), jnp.float32),
    )(adj, h)
    return out
